# baseline (speedup 1.0000x reference)
import jax
import jax.numpy as jnp
from jax.experimental import pallas as pl
from jax.experimental.pallas import tpu as pltpu

L = 2
D = 256
H = 8
FF = 2048
NQ = 300
B = 2
S = 4096
TOPK = 100
DH = D // H
_SCALE = 1.0 / (DH ** 0.5)
_F32 = jnp.float32


def _layer_norm(x, g, b, eps=1e-5):
    m = jnp.mean(x, axis=-1, keepdims=True)
    v = jnp.mean((x - m) ** 2, axis=-1, keepdims=True)
    return (x - m) / jnp.sqrt(v + eps) * g + b


def _bf(a):
    return a.astype(jnp.bfloat16)


def _dot(a, b):
    return jnp.dot(_bf(a), _bf(b), preferred_element_type=_F32)


def _dot_t(a, b):
    return jax.lax.dot_general(_bf(a), _bf(b), (((1,), (1,)), ((), ())),
                               preferred_element_type=_F32)


def _build_mask(pcr_ref, pcc_ref):
    def boxes_r():
        cx = jax.nn.sigmoid(pcr_ref[0, :, 0:1])
        cy = jax.nn.sigmoid(pcr_ref[0, :, 1:2])
        w = jax.nn.sigmoid(pcr_ref[0, :, 2:3])
        h = jax.nn.sigmoid(pcr_ref[0, :, 3:4])
        return cx - 0.5 * w, cy - 0.5 * h, cx + 0.5 * w, cy + 0.5 * h

    def boxes_c():
        cx = jax.nn.sigmoid(pcc_ref[0, 0:1, :])
        cy = jax.nn.sigmoid(pcc_ref[0, 1:2, :])
        w = jax.nn.sigmoid(pcc_ref[0, 2:3, :])
        h = jax.nn.sigmoid(pcc_ref[0, 3:4, :])
        return cx - 0.5 * w, cy - 0.5 * h, cx + 0.5 * w, cy + 0.5 * h

    x0r, y0r, x1r, y1r = boxes_r()
    x0c, y0c, x1c, y1c = boxes_c()
    a_r = (x1r - x0r) * (y1r - y0r)
    a_c = (x1c - x0c) * (y1c - y0c)
    wi = jnp.clip(jnp.minimum(x1r, x1c) - jnp.maximum(x0r, x0c), 0.0)
    hi = jnp.clip(jnp.minimum(y1r, y1c) - jnp.maximum(y0r, y0c), 0.0)
    inter = wi * hi
    union = a_r + a_c - inter
    iou = inter / union
    wc = jnp.clip(jnp.maximum(x1r, x1c) - jnp.minimum(x0r, x0c), 0.0)
    hc = jnp.clip(jnp.maximum(y1r, y1c) - jnp.minimum(y0r, y0c), 0.0)
    area = wc * hc
    score = 1.0 - (iou - (area - union) / area)

    sbits = jax.lax.bitcast_convert_type(score, jnp.int32)

    def body(_, carry):
        lo, hi_ = carry
        mid = lo + ((hi_ - lo) >> 1)
        cnt = jnp.sum((sbits < mid).astype(jnp.int32), axis=1, keepdims=True)
        big = cnt >= TOPK
        return jnp.where(big, lo, mid), jnp.where(big, mid, hi_)

    lo0 = jnp.zeros((NQ, 1), jnp.int32)
    hi0 = jnp.full((NQ, 1), 0x7f000000, jnp.int32)
    kth, _ = jax.lax.fori_loop(0, 31, body, (lo0, hi0))
    cnt_lt = jnp.sum((sbits < kth).astype(jnp.int32), axis=1, keepdims=True)
    need = (TOPK - cnt_lt).astype(_F32)
    eq = (sbits == kth).astype(_F32)
    tri = (jax.lax.broadcasted_iota(jnp.int32, (NQ, NQ), 0)
           <= jax.lax.broadcasted_iota(jnp.int32, (NQ, NQ), 1)).astype(_F32)
    csum = jax.lax.dot_general(eq, tri, (((1,), (0,)), ((), ())),
                               preferred_element_type=_F32)
    return (sbits < kth) | ((eq > 0) & (csum <= need))


def _kv_kernel(mem_ref, p_ref, wk_ref, bk_ref, wv_ref, bv_ref, k_ref, v_ref):
    mem = mem_ref[...]
    k_ref[0] = _bf(_dot(mem + p_ref[...], wk_ref[...]) + bk_ref[...])
    v_ref[0] = _bf(_dot(mem, wv_ref[...]) + bv_ref[...])


def _sa_kernel(x_ref, qp_ref, mask_ref, wq_ref, bq_ref, wk_ref, bk_ref,
               wv_ref, bv_ref, wo_ref, bo_ref, g_ref, b_ref, o_ref):
    x = x_ref[0]
    qp = qp_ref[...]
    mask = mask_ref[0] > 0
    q = x + qp
    qa = (_dot(q, wq_ref[...]) + bq_ref[...]) * _SCALE
    ka = _dot(q, wk_ref[...]) + bk_ref[...]
    va = _dot(x, wv_ref[...]) + bv_ref[...]
    outs = []
    for h in range(H):
        sl = slice(h * DH, (h + 1) * DH)
        s = _dot_t(qa[:, sl], ka[:, sl])
        s = jnp.where(mask, s, -1e9)
        e = _bf(jnp.exp(s - jnp.max(s, axis=-1, keepdims=True)))
        inv = 1.0 / jnp.sum(e.astype(_F32), axis=-1, keepdims=True)
        outs.append(_dot(e, va[:, sl]) * inv)
    o = _dot(jnp.concatenate(outs, axis=1), wo_ref[...])
    o_ref[0] = _layer_norm(o + bo_ref[...] + x, g_ref[...], b_ref[...])


def _ca_attn_kernel(x_ref, qp_ref, k_ref, v_ref, wq_ref, bq_ref, wo_ref,
                    bo_ref, g_ref, b_ref, o_ref, attn_ref):
    x = x_ref[0]
    qa = (_dot(x + qp_ref[...], wq_ref[...]) + bq_ref[...]) * _SCALE
    ka = k_ref[0]
    va = v_ref[0]
    outs = []
    for h in range(H):
        sl = slice(h * DH, (h + 1) * DH)
        s = _dot_t(qa[:, sl], ka[:, sl])
        e = _bf(jnp.exp(s - jnp.max(s, axis=-1, keepdims=True)))
        inv = 1.0 / jnp.sum(e.astype(_F32), axis=-1, keepdims=True)
        pr = e * (inv * (1.0 / H))
        if h == 0:
            attn_ref[0] = pr
        else:
            attn_ref[0] += pr
        outs.append(_dot(e, va[:, sl]) * inv)
    o = _dot(jnp.concatenate(outs, axis=1), wo_ref[...])
    o_ref[0] = _layer_norm(o + bo_ref[...] + x, g_ref[...], b_ref[...])


def _ffn_final_kernel(x_ref, w1_ref, b1_ref, w2_ref, b2_ref, g_ref, b_ref,
                      ng_ref, nb_ref, o_ref):
    x = x_ref[0]
    hmid = jnp.maximum(_dot(x, w1_ref[...]) + b1_ref[...], 0.0)
    y = _dot(hmid, w2_ref[...]) + b2_ref[...] + x
    y = _layer_norm(y, g_ref[...], b_ref[...])
    o_ref[0] = _layer_norm(y, ng_ref[...], nb_ref[...])


def _make_layer_kernel(first, last):
    def _layer_kernel(*refs):
        it = iter(refs)
        if first:
            pcr_ref = next(it)
            pcc_ref = next(it)
        x_ref = next(it)
        qp_ref = next(it)
        if not first:
            mask_ref = next(it)
        k_ref = next(it)
        v_ref = next(it)
        (sa_Wq_ref, sa_bq_ref, sa_Wk_ref, sa_bk_ref, sa_Wv_ref, sa_bv_ref,
         sa_Wo_ref, sa_bo_ref, ca_Wq_ref, ca_bq_ref, ca_Wo_ref, ca_bo_ref,
         ffn_W1_ref, ffn_b1_ref, ffn_W2_ref, ffn_b2_ref, ln1_g_ref,
         ln1_b_ref, ln2_g_ref, ln2_b_ref, ln3_g_ref, ln3_b_ref) = \
            (next(it) for _ in range(22))
        if last:
            norm_g_ref = next(it)
            norm_b_ref = next(it)
        o_ref = next(it)
        if first:
            mask_out_ref = next(it)
        if last:
            attn_ref = next(it)

        if first:
            mask = _build_mask(pcr_ref, pcc_ref)
            mask_out_ref[0] = mask.astype(_F32)
        else:
            mask = mask_ref[0] > 0
        x = x_ref[...] if first else x_ref[0]
        qp = qp_ref[...]

        q = x + qp
        qa = (_dot(q, sa_Wq_ref[...]) + sa_bq_ref[...]) * _SCALE
        ka = _dot(q, sa_Wk_ref[...]) + sa_bk_ref[...]
        va = _dot(x, sa_Wv_ref[...]) + sa_bv_ref[...]
        outs = []
        for h in range(H):
            sl = slice(h * DH, (h + 1) * DH)
            s = _dot_t(qa[:, sl], ka[:, sl])
            s = jnp.where(mask, s, -1e9)
            e = _bf(jnp.exp(s - jnp.max(s, axis=-1, keepdims=True)))
            inv = 1.0 / jnp.sum(e.astype(_F32), axis=-1, keepdims=True)
            outs.append(_dot(e, va[:, sl]) * inv)
        o = _dot(jnp.concatenate(outs, axis=1), sa_Wo_ref[...])
        x = _layer_norm(o + sa_bo_ref[...] + x, ln1_g_ref[...], ln1_b_ref[...])

        qa = (_dot(x + qp, ca_Wq_ref[...]) + ca_bq_ref[...]) * _SCALE
        ka = k_ref[0]
        va = v_ref[0]
        outs = []
        for h in range(H):
            sl = slice(h * DH, (h + 1) * DH)
            s = _dot_t(qa[:, sl], ka[:, sl])
            e = _bf(jnp.exp(s - jnp.max(s, axis=-1, keepdims=True)))
            inv = 1.0 / jnp.sum(e.astype(_F32), axis=-1, keepdims=True)
            if last:
                pr = e * (inv * (1.0 / H))
                if h == 0:
                    attn_ref[0] = pr
                else:
                    attn_ref[0] += pr
            outs.append(_dot(e, va[:, sl]) * inv)
        o = _dot(jnp.concatenate(outs, axis=1), ca_Wo_ref[...])
        x = _layer_norm(o + ca_bo_ref[...] + x, ln2_g_ref[...], ln2_b_ref[...])

        hmid = jnp.maximum(_dot(x, ffn_W1_ref[...]) + ffn_b1_ref[...], 0.0)
        y = _dot(hmid, ffn_W2_ref[...]) + ffn_b2_ref[...] + x
        x = _layer_norm(y, ln3_g_ref[...], ln3_b_ref[...])
        if last:
            x = _layer_norm(x, norm_g_ref[...], norm_b_ref[...])
        o_ref[0] = x
    return _layer_kernel


def _bspec(shape, index_map):
    return pl.BlockSpec(shape, index_map)


def _row(v):
    return v.reshape(1, -1)


def kernel(tgt, memory, pos, query_pos, pos_centers, sa_Wq, sa_bq, sa_Wk,
           sa_bk, sa_Wv, sa_bv, sa_Wo, sa_bo, ca_Wq, ca_bq, ca_Wk, ca_bk,
           ca_Wv, ca_bv, ca_Wo, ca_bo, ffn_W1, ffn_b1, ffn_W2, ffn_b2,
           ln1_g, ln1_b, ln2_g, ln2_b, ln3_g, ln3_b, norm_g, norm_b):
    f32 = _F32
    bf16 = jnp.bfloat16
    tgt2 = tgt.reshape(NQ, B * D)
    qp2 = query_pos.reshape(NQ, B * D)
    mem2 = memory.reshape(S, B * D)
    pos2 = pos.reshape(S, B * D)
    pcr = pos_centers.transpose(1, 0, 2)
    pcc = pos_centers.transpose(1, 2, 0)

    wk_cat = _bf(jnp.concatenate([ca_Wk[0], ca_Wk[1]], axis=1))
    bk_cat = _row(jnp.concatenate([ca_bk[0], ca_bk[1]]))
    wv_cat = _bf(jnp.concatenate([ca_Wv[0], ca_Wv[1]], axis=1))
    bv_cat = _row(jnp.concatenate([ca_bv[0], ca_bv[1]]))
    SCNK = 4
    kall, vall = pl.pallas_call(
        _kv_kernel,
        grid=(B, SCNK),
        in_specs=[_bspec((S // SCNK, D), lambda b, i: (i, b)),
                  _bspec((S // SCNK, D), lambda b, i: (i, b)),
                  _bspec((D, 2 * D), lambda b, i: (0, 0)),
                  _bspec((1, 2 * D), lambda b, i: (0, 0)),
                  _bspec((D, 2 * D), lambda b, i: (0, 0)),
                  _bspec((1, 2 * D), lambda b, i: (0, 0))],
        out_specs=[_bspec((1, S // SCNK, 2 * D), lambda b, i: (b, i, 0)),
                   _bspec((1, S // SCNK, 2 * D), lambda b, i: (b, i, 0))],
        out_shape=[jax.ShapeDtypeStruct((B, S, 2 * D), bf16),
                   jax.ShapeDtypeStruct((B, S, 2 * D), bf16)],
    )(mem2, pos2, wk_cat, bk_cat, wv_cat, bv_cat)

    xspec = _bspec((NQ, D), lambda b: (0, b))
    bspec3 = _bspec((1, NQ, D), lambda b: (b, 0, 0))
    mspec = _bspec((1, NQ, NQ), lambda b: (b, 0, 0))

    def _w(shape):
        return _bspec(shape, lambda b: tuple(0 for _ in shape))

    def _weights(l):
        return (_bf(sa_Wq[l]), _row(sa_bq[l]), _bf(sa_Wk[l]),
                _row(sa_bk[l]), _bf(sa_Wv[l]), _row(sa_bv[l]),
                _bf(sa_Wo[l]), _row(sa_bo[l]), _bf(ca_Wq[l]),
                _row(ca_bq[l]), _bf(ca_Wo[l]), _row(ca_bo[l]),
                _bf(ffn_W1[l]), _row(ffn_b1[l]), _bf(ffn_W2[l]),
                _row(ffn_b2[l]), _row(ln1_g[l]), _row(ln1_b[l]),
                _row(ln2_g[l]), _row(ln2_b[l]), _row(ln3_g[l]),
                _row(ln3_b[l]))

    wspecs = [_w((D, D)), _w((1, D)), _w((D, D)), _w((1, D)),
              _w((D, D)), _w((1, D)), _w((D, D)), _w((1, D)),
              _w((D, D)), _w((1, D)), _w((D, D)), _w((1, D)),
              _w((D, FF)), _w((1, FF)), _w((FF, D)), _w((1, D)),
              _w((1, D)), _w((1, D)), _w((1, D)), _w((1, D)),
              _w((1, D)), _w((1, D))]
    kvspec = [_bspec((1, S, D), lambda b, l=l: (b, 0, l)) for l in range(L)]

    x1, mask = pl.pallas_call(
        _make_layer_kernel(first=True, last=False),
        grid=(B,),
        in_specs=[_bspec((1, NQ, 4), lambda b: (b, 0, 0)),
                  _bspec((1, 4, NQ), lambda b: (b, 0, 0)),
                  xspec, xspec, kvspec[0], kvspec[0]] + wspecs,
        out_specs=[bspec3, mspec],
        out_shape=[jax.ShapeDtypeStruct((B, NQ, D), f32),
                   jax.ShapeDtypeStruct((B, NQ, NQ), f32)],
        compiler_params=pltpu.CompilerParams(
            vmem_limit_bytes=63 * 1024 * 1024),
    )(pcr, pcc, tgt2, qp2, kall, vall, *_weights(0))

    wd = _w((D, D))
    wr = _w((1, D))
    x2 = pl.pallas_call(
        _sa_kernel,
        grid=(B,),
        in_specs=[bspec3, xspec, mspec, wd, wr, wd, wr, wd, wr, wd, wr,
                  wr, wr],
        out_specs=bspec3,
        out_shape=jax.ShapeDtypeStruct((B, NQ, D), f32),
    )(x1, qp2, mask, _bf(sa_Wq[1]), _row(sa_bq[1]), _bf(sa_Wk[1]),
      _row(sa_bk[1]), _bf(sa_Wv[1]), _row(sa_bv[1]), _bf(sa_Wo[1]),
      _row(sa_bo[1]), _row(ln1_g[1]), _row(ln1_b[1]))

    x3, attn = pl.pallas_call(
        _ca_attn_kernel,
        grid=(B,),
        in_specs=[bspec3, xspec, kvspec[1], kvspec[1], wd, wr, wd, wr,
                  wr, wr],
        out_specs=[bspec3, _bspec((1, NQ, S), lambda b: (b, 0, 0))],
        out_shape=[jax.ShapeDtypeStruct((B, NQ, D), f32),
                   jax.ShapeDtypeStruct((B, NQ, S), f32)],
        compiler_params=pltpu.CompilerParams(
            vmem_limit_bytes=63 * 1024 * 1024),
    )(x2, qp2, kall, vall, _bf(ca_Wq[1]), _row(ca_bq[1]), _bf(ca_Wo[1]),
      _row(ca_bo[1]), _row(ln2_g[1]), _row(ln2_b[1]))

    out = pl.pallas_call(
        _ffn_final_kernel,
        grid=(B,),
        in_specs=[bspec3, _w((D, FF)), _w((1, FF)), _w((FF, D)), wr,
                  wr, wr, wr, wr],
        out_specs=bspec3,
        out_shape=jax.ShapeDtypeStruct((B, NQ, D), f32),
    )(x3, _bf(ffn_W1[1]), _row(ffn_b1[1]), _bf(ffn_W2[1]),
      _row(ffn_b2[1]), _row(ln3_g[1]), _row(ln3_b[1]),
      _row(norm_g), _row(norm_b))

    return (out.transpose(1, 0, 2), pos_centers, attn)


# speedup vs baseline: 1.0116x; 1.0116x over previous
import jax
import jax.numpy as jnp
from jax.experimental import pallas as pl
from jax.experimental.pallas import tpu as pltpu

L = 2
D = 256
H = 8
FF = 2048
NQ = 300
B = 2
S = 4096
TOPK = 100
DH = D // H
_SCALE = 1.0 / (DH ** 0.5)
_F32 = jnp.float32


def _layer_norm(x, g, b, eps=1e-5):
    m = jnp.mean(x, axis=-1, keepdims=True)
    v = jnp.mean((x - m) ** 2, axis=-1, keepdims=True)
    return (x - m) / jnp.sqrt(v + eps) * g + b


def _bf(a):
    return a.astype(jnp.bfloat16)


def _dot(a, b):
    return jnp.dot(_bf(a), _bf(b), preferred_element_type=_F32)


def _dot_t(a, b):
    return jax.lax.dot_general(_bf(a), _bf(b), (((1,), (1,)), ((), ())),
                               preferred_element_type=_F32)


def _build_mask(pcr_ref, pcc_ref):
    def boxes_r():
        cx = jax.nn.sigmoid(pcr_ref[0, :, 0:1])
        cy = jax.nn.sigmoid(pcr_ref[0, :, 1:2])
        w = jax.nn.sigmoid(pcr_ref[0, :, 2:3])
        h = jax.nn.sigmoid(pcr_ref[0, :, 3:4])
        return cx - 0.5 * w, cy - 0.5 * h, cx + 0.5 * w, cy + 0.5 * h

    def boxes_c():
        cx = jax.nn.sigmoid(pcc_ref[0, 0:1, :])
        cy = jax.nn.sigmoid(pcc_ref[0, 1:2, :])
        w = jax.nn.sigmoid(pcc_ref[0, 2:3, :])
        h = jax.nn.sigmoid(pcc_ref[0, 3:4, :])
        return cx - 0.5 * w, cy - 0.5 * h, cx + 0.5 * w, cy + 0.5 * h

    x0r, y0r, x1r, y1r = boxes_r()
    x0c, y0c, x1c, y1c = boxes_c()
    a_r = (x1r - x0r) * (y1r - y0r)
    a_c = (x1c - x0c) * (y1c - y0c)
    wi = jnp.clip(jnp.minimum(x1r, x1c) - jnp.maximum(x0r, x0c), 0.0)
    hi = jnp.clip(jnp.minimum(y1r, y1c) - jnp.maximum(y0r, y0c), 0.0)
    inter = wi * hi
    union = a_r + a_c - inter
    iou = inter / union
    wc = jnp.clip(jnp.maximum(x1r, x1c) - jnp.minimum(x0r, x0c), 0.0)
    hc = jnp.clip(jnp.maximum(y1r, y1c) - jnp.minimum(y0r, y0c), 0.0)
    area = wc * hc
    score = 1.0 - (iou - (area - union) / area)

    sbits = jax.lax.bitcast_convert_type(score, jnp.int32)

    def body(_, carry):
        lo, hi_ = carry
        mid = lo + ((hi_ - lo) >> 1)
        cnt = jnp.sum((sbits < mid).astype(jnp.int32), axis=1, keepdims=True)
        big = cnt >= TOPK
        return jnp.where(big, lo, mid), jnp.where(big, mid, hi_)

    lo0 = jnp.zeros((NQ, 1), jnp.int32)
    hi0 = jnp.full((NQ, 1), 0x7f000000, jnp.int32)
    kth, _ = jax.lax.fori_loop(0, 31, body, (lo0, hi0))
    cnt_lt = jnp.sum((sbits < kth).astype(jnp.int32), axis=1, keepdims=True)
    need = (TOPK - cnt_lt).astype(_F32)
    eq = (sbits == kth).astype(_F32)
    tri = (jax.lax.broadcasted_iota(jnp.int32, (NQ, NQ), 0)
           <= jax.lax.broadcasted_iota(jnp.int32, (NQ, NQ), 1)).astype(_F32)
    csum = jax.lax.dot_general(eq, tri, (((1,), (0,)), ((), ())),
                               preferred_element_type=_F32)
    return (sbits < kth) | ((eq > 0) & (csum <= need))


def _kv_kernel(mem_ref, p_ref, wk_ref, bk_ref, wv_ref, bv_ref, k_ref, v_ref):
    mem = mem_ref[...]
    k_ref[0] = _bf(_dot(mem + p_ref[...], wk_ref[...]) + bk_ref[...])
    v_ref[0] = _bf(_dot(mem, wv_ref[...]) + bv_ref[...])


def _sa_kernel(x_ref, qp_ref, mask_ref, wq_ref, bq_ref, wk_ref, bk_ref,
               wv_ref, bv_ref, wo_ref, bo_ref, g_ref, b_ref, o_ref):
    x = x_ref[0]
    qp = qp_ref[...]
    mask = mask_ref[0] > 0
    q = x + qp
    qa = (_dot(q, wq_ref[...]) + bq_ref[...]) * _SCALE
    ka = _dot(q, wk_ref[...]) + bk_ref[...]
    va = _dot(x, wv_ref[...]) + bv_ref[...]
    outs = []
    for h in range(H):
        sl = slice(h * DH, (h + 1) * DH)
        s = _dot_t(qa[:, sl], ka[:, sl])
        s = jnp.where(mask, s, -1e9)
        e = jnp.exp(s - jnp.max(s, axis=-1, keepdims=True))
        inv = 1.0 / jnp.sum(e, axis=-1, keepdims=True)
        outs.append(_dot(e, va[:, sl]) * inv)
    o = _dot(jnp.concatenate(outs, axis=1), wo_ref[...])
    o_ref[0] = _layer_norm(o + bo_ref[...] + x, g_ref[...], b_ref[...])


def _ca_attn_kernel(x_ref, qp_ref, k_ref, v_ref, wq_ref, bq_ref, wo_ref,
                    bo_ref, g_ref, b_ref, o_ref, attn_ref):
    x = x_ref[0]
    qa = (_dot(x + qp_ref[...], wq_ref[...]) + bq_ref[...]) * _SCALE
    ka = k_ref[0]
    va = v_ref[0]
    outs = []
    for h in range(H):
        sl = slice(h * DH, (h + 1) * DH)
        s = _dot_t(qa[:, sl], ka[:, sl])
        e = jnp.exp(s - jnp.max(s, axis=-1, keepdims=True))
        inv = 1.0 / jnp.sum(e, axis=-1, keepdims=True)
        pr = e * (inv * (1.0 / H))
        if h == 0:
            attn_ref[0] = pr
        else:
            attn_ref[0] += pr
        outs.append(_dot(e, va[:, sl]) * inv)
    o = _dot(jnp.concatenate(outs, axis=1), wo_ref[...])
    o_ref[0] = _layer_norm(o + bo_ref[...] + x, g_ref[...], b_ref[...])


def _ffn_final_kernel(x_ref, w1_ref, b1_ref, w2_ref, b2_ref, g_ref, b_ref,
                      ng_ref, nb_ref, o_ref):
    x = x_ref[0]
    hmid = jnp.maximum(_dot(x, w1_ref[...]) + b1_ref[...], 0.0)
    y = _dot(hmid, w2_ref[...]) + b2_ref[...] + x
    y = _layer_norm(y, g_ref[...], b_ref[...])
    o_ref[0] = _layer_norm(y, ng_ref[...], nb_ref[...])


def _make_layer_kernel(first, last):
    def _layer_kernel(*refs):
        it = iter(refs)
        if first:
            pcr_ref = next(it)
            pcc_ref = next(it)
        x_ref = next(it)
        qp_ref = next(it)
        if not first:
            mask_ref = next(it)
        k_ref = next(it)
        v_ref = next(it)
        (sa_Wq_ref, sa_bq_ref, sa_Wk_ref, sa_bk_ref, sa_Wv_ref, sa_bv_ref,
         sa_Wo_ref, sa_bo_ref, ca_Wq_ref, ca_bq_ref, ca_Wo_ref, ca_bo_ref,
         ffn_W1_ref, ffn_b1_ref, ffn_W2_ref, ffn_b2_ref, ln1_g_ref,
         ln1_b_ref, ln2_g_ref, ln2_b_ref, ln3_g_ref, ln3_b_ref) = \
            (next(it) for _ in range(22))
        if last:
            norm_g_ref = next(it)
            norm_b_ref = next(it)
        o_ref = next(it)
        if first:
            mask_out_ref = next(it)
        if last:
            attn_ref = next(it)

        if first:
            mask = _build_mask(pcr_ref, pcc_ref)
            mask_out_ref[0] = mask.astype(_F32)
        else:
            mask = mask_ref[0] > 0
        x = x_ref[...] if first else x_ref[0]
        qp = qp_ref[...]

        q = x + qp
        qa = (_dot(q, sa_Wq_ref[...]) + sa_bq_ref[...]) * _SCALE
        ka = _dot(q, sa_Wk_ref[...]) + sa_bk_ref[...]
        va = _dot(x, sa_Wv_ref[...]) + sa_bv_ref[...]
        outs = []
        for h in range(H):
            sl = slice(h * DH, (h + 1) * DH)
            s = _dot_t(qa[:, sl], ka[:, sl])
            s = jnp.where(mask, s, -1e9)
            e = jnp.exp(s - jnp.max(s, axis=-1, keepdims=True))
            inv = 1.0 / jnp.sum(e, axis=-1, keepdims=True)
            outs.append(_dot(e, va[:, sl]) * inv)
        o = _dot(jnp.concatenate(outs, axis=1), sa_Wo_ref[...])
        x = _layer_norm(o + sa_bo_ref[...] + x, ln1_g_ref[...], ln1_b_ref[...])

        qa = (_dot(x + qp, ca_Wq_ref[...]) + ca_bq_ref[...]) * _SCALE
        ka = k_ref[0]
        va = v_ref[0]
        outs = []
        for h in range(H):
            sl = slice(h * DH, (h + 1) * DH)
            s = _dot_t(qa[:, sl], ka[:, sl])
            e = jnp.exp(s - jnp.max(s, axis=-1, keepdims=True))
            inv = 1.0 / jnp.sum(e, axis=-1, keepdims=True)
            if last:
                pr = e * (inv * (1.0 / H))
                if h == 0:
                    attn_ref[0] = pr
                else:
                    attn_ref[0] += pr
            outs.append(_dot(e, va[:, sl]) * inv)
        o = _dot(jnp.concatenate(outs, axis=1), ca_Wo_ref[...])
        x = _layer_norm(o + ca_bo_ref[...] + x, ln2_g_ref[...], ln2_b_ref[...])

        hmid = jnp.maximum(_dot(x, ffn_W1_ref[...]) + ffn_b1_ref[...], 0.0)
        y = _dot(hmid, ffn_W2_ref[...]) + ffn_b2_ref[...] + x
        x = _layer_norm(y, ln3_g_ref[...], ln3_b_ref[...])
        if last:
            x = _layer_norm(x, norm_g_ref[...], norm_b_ref[...])
        o_ref[0] = x
    return _layer_kernel


def _bspec(shape, index_map):
    return pl.BlockSpec(shape, index_map)


def _row(v):
    return v.reshape(1, -1)


def kernel(tgt, memory, pos, query_pos, pos_centers, sa_Wq, sa_bq, sa_Wk,
           sa_bk, sa_Wv, sa_bv, sa_Wo, sa_bo, ca_Wq, ca_bq, ca_Wk, ca_bk,
           ca_Wv, ca_bv, ca_Wo, ca_bo, ffn_W1, ffn_b1, ffn_W2, ffn_b2,
           ln1_g, ln1_b, ln2_g, ln2_b, ln3_g, ln3_b, norm_g, norm_b):
    f32 = _F32
    bf16 = jnp.bfloat16
    tgt2 = tgt.reshape(NQ, B * D)
    qp2 = query_pos.reshape(NQ, B * D)
    mem2 = memory.reshape(S, B * D)
    pos2 = pos.reshape(S, B * D)
    pcr = pos_centers.transpose(1, 0, 2)
    pcc = pos_centers.transpose(1, 2, 0)

    wk_cat = _bf(jnp.concatenate([ca_Wk[0], ca_Wk[1]], axis=1))
    bk_cat = _row(jnp.concatenate([ca_bk[0], ca_bk[1]]))
    wv_cat = _bf(jnp.concatenate([ca_Wv[0], ca_Wv[1]], axis=1))
    bv_cat = _row(jnp.concatenate([ca_bv[0], ca_bv[1]]))
    SCNK = 4
    kall, vall = pl.pallas_call(
        _kv_kernel,
        grid=(B, SCNK),
        in_specs=[_bspec((S // SCNK, D), lambda b, i: (i, b)),
                  _bspec((S // SCNK, D), lambda b, i: (i, b)),
                  _bspec((D, 2 * D), lambda b, i: (0, 0)),
                  _bspec((1, 2 * D), lambda b, i: (0, 0)),
                  _bspec((D, 2 * D), lambda b, i: (0, 0)),
                  _bspec((1, 2 * D), lambda b, i: (0, 0))],
        out_specs=[_bspec((1, S // SCNK, 2 * D), lambda b, i: (b, i, 0)),
                   _bspec((1, S // SCNK, 2 * D), lambda b, i: (b, i, 0))],
        out_shape=[jax.ShapeDtypeStruct((B, S, 2 * D), bf16),
                   jax.ShapeDtypeStruct((B, S, 2 * D), bf16)],
    )(mem2, pos2, wk_cat, bk_cat, wv_cat, bv_cat)

    xspec = _bspec((NQ, D), lambda b: (0, b))
    bspec3 = _bspec((1, NQ, D), lambda b: (b, 0, 0))
    mspec = _bspec((1, NQ, NQ), lambda b: (b, 0, 0))

    def _w(shape):
        return _bspec(shape, lambda b: tuple(0 for _ in shape))

    def _weights(l):
        return (_bf(sa_Wq[l]), _row(sa_bq[l]), _bf(sa_Wk[l]),
                _row(sa_bk[l]), _bf(sa_Wv[l]), _row(sa_bv[l]),
                _bf(sa_Wo[l]), _row(sa_bo[l]), _bf(ca_Wq[l]),
                _row(ca_bq[l]), _bf(ca_Wo[l]), _row(ca_bo[l]),
                _bf(ffn_W1[l]), _row(ffn_b1[l]), _bf(ffn_W2[l]),
                _row(ffn_b2[l]), _row(ln1_g[l]), _row(ln1_b[l]),
                _row(ln2_g[l]), _row(ln2_b[l]), _row(ln3_g[l]),
                _row(ln3_b[l]))

    wspecs = [_w((D, D)), _w((1, D)), _w((D, D)), _w((1, D)),
              _w((D, D)), _w((1, D)), _w((D, D)), _w((1, D)),
              _w((D, D)), _w((1, D)), _w((D, D)), _w((1, D)),
              _w((D, FF)), _w((1, FF)), _w((FF, D)), _w((1, D)),
              _w((1, D)), _w((1, D)), _w((1, D)), _w((1, D)),
              _w((1, D)), _w((1, D))]
    kvspec = [_bspec((1, S, D), lambda b, l=l: (b, 0, l)) for l in range(L)]

    x1, mask = pl.pallas_call(
        _make_layer_kernel(first=True, last=False),
        grid=(B,),
        in_specs=[_bspec((1, NQ, 4), lambda b: (b, 0, 0)),
                  _bspec((1, 4, NQ), lambda b: (b, 0, 0)),
                  xspec, xspec, kvspec[0], kvspec[0]] + wspecs,
        out_specs=[bspec3, mspec],
        out_shape=[jax.ShapeDtypeStruct((B, NQ, D), f32),
                   jax.ShapeDtypeStruct((B, NQ, NQ), f32)],
        compiler_params=pltpu.CompilerParams(
            vmem_limit_bytes=63 * 1024 * 1024),
    )(pcr, pcc, tgt2, qp2, kall, vall, *_weights(0))

    wd = _w((D, D))
    wr = _w((1, D))
    x2 = pl.pallas_call(
        _sa_kernel,
        grid=(B,),
        in_specs=[bspec3, xspec, mspec, wd, wr, wd, wr, wd, wr, wd, wr,
                  wr, wr],
        out_specs=bspec3,
        out_shape=jax.ShapeDtypeStruct((B, NQ, D), f32),
    )(x1, qp2, mask, _bf(sa_Wq[1]), _row(sa_bq[1]), _bf(sa_Wk[1]),
      _row(sa_bk[1]), _bf(sa_Wv[1]), _row(sa_bv[1]), _bf(sa_Wo[1]),
      _row(sa_bo[1]), _row(ln1_g[1]), _row(ln1_b[1]))

    x3, attn = pl.pallas_call(
        _ca_attn_kernel,
        grid=(B,),
        in_specs=[bspec3, xspec, kvspec[1], kvspec[1], wd, wr, wd, wr,
                  wr, wr],
        out_specs=[bspec3, _bspec((1, NQ, S), lambda b: (b, 0, 0))],
        out_shape=[jax.ShapeDtypeStruct((B, NQ, D), f32),
                   jax.ShapeDtypeStruct((B, NQ, S), f32)],
        compiler_params=pltpu.CompilerParams(
            vmem_limit_bytes=63 * 1024 * 1024),
    )(x2, qp2, kall, vall, _bf(ca_Wq[1]), _row(ca_bq[1]), _bf(ca_Wo[1]),
      _row(ca_bo[1]), _row(ln2_g[1]), _row(ln2_b[1]))

    out = pl.pallas_call(
        _ffn_final_kernel,
        grid=(B,),
        in_specs=[bspec3, _w((D, FF)), _w((1, FF)), _w((FF, D)), wr,
                  wr, wr, wr, wr],
        out_specs=bspec3,
        out_shape=jax.ShapeDtypeStruct((B, NQ, D), f32),
    )(x3, _bf(ffn_W1[1]), _row(ffn_b1[1]), _bf(ffn_W2[1]),
      _row(ffn_b2[1]), _row(ln3_g[1]), _row(ln3_b[1]),
      _row(norm_g), _row(norm_b))

    return (out.transpose(1, 0, 2), pos_centers, attn)
